# Initial kernel scaffold; baseline (speedup 1.0000x reference)
import jax
import jax.numpy as jnp
from jax.experimental import pallas as pl


def kernel(cat_mask, quaternion, scales, xy, z):
    raise NotImplementedError("write your pallas kernel here")


# speedup vs baseline: 14.9113x; 14.9113x over previous
import functools

import jax
import jax.numpy as jnp
import numpy as np
from jax.experimental import pallas as pl

_CLASSES = 9
_CM1 = _CLASSES - 1
_INTR = np.array(
    [[572.4114, 0.0, 325.2611], [0.0, 573.57043, 242.04899], [0.0, 0.0, 1.0]],
    dtype=np.float32,
)
_KINV = np.linalg.inv(_INTR).astype(np.float32)

_B, _H, _W = 8, 224, 224
_HW = _H * _W
_ROWS = _HW // 128
_RT_H = 56
_NHT = _ROWS // _RT_H

_NSLOT = 11
_PS_ROWS = 96


def _gather_body(cat_ref, q_ref, s_ref, xy_ref, z_ref,
                 gq_ref, gs_ref, gxy_ref, gz_ref, ps_ref):
    h = pl.program_id(1)
    cm = cat_ref[0]
    idx = jnp.clip(cm - 1, 0, _CM1 - 1)
    fg = cm > 0

    @pl.when(h == 0)
    def _():
        ps_ref[...] = jnp.zeros((1, _PS_ROWS, 128), jnp.float32)

    fields = ((q_ref, gq_ref, 4, 0), (s_ref, gs_ref, 3, 4),
              (xy_ref, gxy_ref, 2, 7), (z_ref, None, 1, 9))

    for c in range(_CM1):
        m = jnp.where((idx == c) & fg, 1.0, 0.0)
        r = 10 * 8 + c
        ps_ref[0, pl.ds(r, 1), :] = ps_ref[0, pl.ds(r, 1), :] + jnp.sum(
            m, axis=0, keepdims=True)
        for in_ref, out_ref, nch, slot0 in fields:
            for ch in range(nch):
                p = m * in_ref[0, c * nch + ch]
                r = (slot0 + ch) * 8 + c
                ps_ref[0, pl.ds(r, 1), :] = ps_ref[0, pl.ds(r, 1), :] + jnp.sum(
                    p, axis=0, keepdims=True)
                if out_ref is None:
                    if c == 0:
                        gz_ref[0] = p
                    else:
                        gz_ref[0] = gz_ref[0] + p
                else:
                    if c == 0:
                        out_ref[0, ch] = p
                    else:
                        out_ref[0, ch] = out_ref[0, ch] + p


def _epilogue_body(ps_ref, out_ref):
    S = jnp.sum(ps_ref[...], axis=2)
    cnt = S[:, 80:88]
    denom = jnp.maximum(cnt, 1.0)
    q0 = S[:, 0:8] / denom
    q1 = S[:, 8:16] / denom
    q2 = S[:, 16:24] / denom
    q3 = S[:, 24:32] / denom
    s0 = S[:, 32:40] / denom
    s1 = S[:, 40:48] / denom
    s2 = S[:, 48:56] / denom
    x0 = S[:, 56:64] / denom
    x1 = S[:, 64:72] / denom
    zm = S[:, 72:80] / denom
    nrm = jnp.maximum(jnp.sqrt(q0 * q0 + q1 * q1 + q2 * q2 + q3 * q3), 1e-8)
    qw, qx, qy, qz = q0 / nrm, q1 / nrm, q2 / nrm, q3 / nrm
    r00 = 1 - 2 * (qy * qy + qz * qz)
    r01 = 2 * (qx * qy - qz * qw)
    r02 = 2 * (qx * qz + qy * qw)
    r10 = 2 * (qx * qy + qz * qw)
    r11 = 1 - 2 * (qx * qx + qz * qz)
    r12 = 2 * (qy * qz - qx * qw)
    r20 = 2 * (qx * qz - qy * qw)
    r21 = 2 * (qy * qz + qx * qw)
    r22 = 1 - 2 * (qx * qx + qy * qy)
    zval = jnp.exp(zm)
    t0 = zval * (x0 * _KINV[0, 0] + x1 * _KINV[0, 1] + _KINV[0, 2])
    t1 = zval * (x0 * _KINV[1, 0] + x1 * _KINV[1, 1] + _KINV[1, 2])
    t2 = zval * (x0 * _KINV[2, 0] + x1 * _KINV[2, 1] + _KINV[2, 2])
    one = jnp.ones_like(q0)
    zero = jnp.zeros_like(q0)
    rows = [q0, q1, q2, q3, s0, s1, s2, x0, x1, zm, cnt,
            r00, r01, r02, t0, r10, r11, r12, t1, r20, r21, r22, t2,
            zero, zero, zero, one,
            zero, zero, zero, zero, zero]
    out_ref[...] = jnp.stack(rows, axis=0)


@functools.partial(jax.jit, static_argnums=())
def kernel(cat_mask, quaternion, scales, xy, z):
    B, Hh, Ww = cat_mask.shape
    cm = cat_mask.reshape(B, _ROWS, 128).astype(jnp.int32)
    q = quaternion.reshape(B, 4 * _CM1, _ROWS, 128)
    s = scales.reshape(B, 3 * _CM1, _ROWS, 128)
    x = xy.reshape(B, 2 * _CM1, _ROWS, 128)
    zz = z.reshape(B, _CM1, _ROWS, 128)

    grid = (B, _NHT)
    out_shapes = (
        jax.ShapeDtypeStruct((B, 4, _ROWS, 128), jnp.float32),
        jax.ShapeDtypeStruct((B, 3, _ROWS, 128), jnp.float32),
        jax.ShapeDtypeStruct((B, 2, _ROWS, 128), jnp.float32),
        jax.ShapeDtypeStruct((B, _ROWS, 128), jnp.float32),
        jax.ShapeDtypeStruct((B, _PS_ROWS, 128), jnp.float32),
    )
    in_specs = [
        pl.BlockSpec((1, _RT_H, 128), lambda b, h: (b, h, 0)),
        pl.BlockSpec((1, 4 * _CM1, _RT_H, 128), lambda b, h: (b, 0, h, 0)),
        pl.BlockSpec((1, 3 * _CM1, _RT_H, 128), lambda b, h: (b, 0, h, 0)),
        pl.BlockSpec((1, 2 * _CM1, _RT_H, 128), lambda b, h: (b, 0, h, 0)),
        pl.BlockSpec((1, _CM1, _RT_H, 128), lambda b, h: (b, 0, h, 0)),
    ]
    out_specs = (
        pl.BlockSpec((1, 4, _RT_H, 128), lambda b, h: (b, 0, h, 0)),
        pl.BlockSpec((1, 3, _RT_H, 128), lambda b, h: (b, 0, h, 0)),
        pl.BlockSpec((1, 2, _RT_H, 128), lambda b, h: (b, 0, h, 0)),
        pl.BlockSpec((1, _RT_H, 128), lambda b, h: (b, h, 0)),
        pl.BlockSpec((1, _PS_ROWS, 128), lambda b, h: (b, 0, 0)),
    )
    gq, gs, gxy, gz, psums = pl.pallas_call(
        _gather_body,
        grid=grid,
        in_specs=in_specs,
        out_specs=out_specs,
        out_shape=out_shapes,
    )(cm, q, s, x, zz)

    E = pl.pallas_call(
        _epilogue_body,
        out_shape=jax.ShapeDtypeStruct((32, 8, 8), jnp.float32),
    )(psums)

    def col(r):
        return E[r].T.reshape(_CM1 * B)

    aq = jnp.stack([col(0), col(1), col(2), col(3)], axis=1)
    ascl = jnp.stack([col(4), col(5), col(6)], axis=1)
    axy = jnp.stack([col(7), col(8)], axis=1)
    az = col(9)[:, None]
    fg_counts = col(10)[:, None]
    RT = jnp.stack([col(11 + i) for i in range(16)], axis=1).reshape(
        _CM1 * B, 4, 4)

    gq = gq.reshape(B, 4, _H, _W)
    gs = gs.reshape(B, 3, _H, _W)
    gxy = gxy.reshape(B, 2, _H, _W)
    gz = gz.reshape(B, _H, _W)
    return aq, ascl, axy, az, RT, fg_counts, gq, gs, gxy, gz
